# Initial kernel scaffold; baseline (speedup 1.0000x reference)
import jax
import jax.numpy as jnp
from jax.experimental import pallas as pl


def kernel(x, edge_index, batch, emb0, emb1, emb2, emb5, W1, b1, W2, b2, W3, b3, g1, be1, g2, be2, g3, be3, Wfc, bfc):
    raise NotImplementedError("write your pallas kernel here")


# speedup vs baseline: 3.2313x; 3.2313x over previous
import jax
import jax.numpy as jnp
from jax import lax
from jax.experimental import pallas as pl
from jax.experimental.pallas import tpu as pltpu
from jax.experimental.pallas import tpu_sc as plsc

N = 50000
E = 800000
G = 512
HID = 128

NC = 2
NS = 16
LANES = 128

N_PAD = 50176
NPT = N_PAD // NS
E_ROWS = 6400
E_PAD = E_ROWS * LANES
RPT = E_ROWS // NS
EIB = 4
EPAIRS = RPT // (2 * EIB)
DRPT = (E_ROWS // NC) // NS
DIB = 8
DBLKS = DRPT // DIB
ARPT = NPT * 4 // LANES
AIB = 7
APAIRS = ARPT // (2 * AIB)
NBLK = N_PAD // 512
CH = 16
CW = 8
CPC = CH // NC
N_ACC = 50016
NACC_T = N_ACC // NS
TROWS = 480

_f32 = jnp.float32
_i32 = jnp.int32


def _sc_emb_deg_body(x_hbm, dst_hbm, tcat_hbm, z2d_hbm, ones_hbm,
                     xw1_hbm, degp_hbm,
                     xbuf, gidx, nidx, dbuf, obuf, gbuf0, gbuf1,
                     acc, sem_g, sem_s):
    c = lax.axis_index("c")
    s = lax.axis_index("s")
    nbase = s * NPT

    pltpu.sync_copy(ones_hbm, obuf)

    pltpu.sync_copy(x_hbm.at[pl.ds(nbase, NPT)], xbuf)
    lane = lax.iota(_i32, 16)

    def build_row(r):
        for half in range(2):
            rows16 = (2 * r + half) * 16 + lane
            nid = jnp.minimum(nbase + rows16, N)
            for t, (col, off) in enumerate(((1, 0), (2, 96), (3, 192), (4, 288))):
                vals = plsc.load_gather(xbuf, [rows16, jnp.full((16,), col, _i32)])
                iv = vals.astype(_i32) + off
                gidx[r, pl.ds(32 * t + 16 * half, 16)] = iv
                nidx[r, pl.ds(32 * t + 16 * half, 16)] = nid

    pl.loop(0, ARPT)(build_row)

    if True:
        pltpu.sync_copy(z2d_hbm, acc.at[pl.ds(s * NACC_T, NACC_T)])
        plsc.subcore_barrier()
        drow0 = (c * NS + s) * DRPT

        def deg_block(b):
            pltpu.sync_copy(dst_hbm.at[pl.ds(drow0 + b * DIB, DIB)], dbuf)
            for j in range(DIB):
                pltpu.async_copy(obuf, acc.at[dbuf.at[j]], sem_s, add=True)
            for j in range(DIB):
                pltpu.make_async_copy(ones_hbm, obuf, sem_s).wait()

        pl.loop(0, DBLKS)(deg_block)
        plsc.subcore_barrier()
        pltpu.sync_copy(acc.at[pl.ds(s * NACC_T, NACC_T)],
                        degp_hbm.at[c, pl.ds(s * NACC_T, NACC_T)])
        plsc.subcore_barrier()

        for fc in range(CPC):
            f = CPC * c + fc
            pltpu.sync_copy(z2d_hbm, acc.at[pl.ds(s * NACC_T, NACC_T)])
            plsc.subcore_barrier()

            def chunk_pair(k, f=f):
                for par, gb in ((0, gbuf0), (1, gbuf1)):
                    b = 2 * k + par

                    @pl.when(k >= 1)
                    def _drain():
                        for j in range(AIB):
                            pltpu.make_async_copy(
                                tcat_hbm.at[0, pl.ds(0, LANES)], gb.at[j],
                                sem_s).wait()

                    descs = []
                    for j in range(AIB):
                        descs.append(pltpu.async_copy(
                            tcat_hbm.at[f].at[gidx.at[b * AIB + j]],
                            gb.at[j], sem_g))
                    for d in descs:
                        d.wait()
                    for j in range(AIB):
                        pltpu.async_copy(gb.at[j], acc.at[nidx.at[b * AIB + j]],
                                         sem_s, add=True)

            pl.loop(0, APAIRS)(chunk_pair)
            for gb in (gbuf0, gbuf1):
                for j in range(AIB):
                    pltpu.make_async_copy(
                        tcat_hbm.at[0, pl.ds(0, LANES)], gb.at[j], sem_s).wait()
            plsc.subcore_barrier()
            pltpu.sync_copy(acc.at[pl.ds(s * NACC_T, NACC_T)],
                            xw1_hbm.at[f, pl.ds(s * NACC_T, NACC_T)])
            plsc.subcore_barrier()


def _sc_emb_deg(xp, dst2, tcat_c, z2d, ones2d):
    mesh = plsc.VectorSubcoreMesh(core_axis_name="c", subcore_axis_name="s")
    return pl.kernel(
        _sc_emb_deg_body,
        out_type=(jax.ShapeDtypeStruct((CH, N_PAD, CW), _f32),
                  jax.ShapeDtypeStruct((NC, N_PAD, CW), _f32)),
        mesh=mesh,
        compiler_params=pltpu.CompilerParams(needs_layout_passes=False,
                                             use_tc_tiling_on_sc=False),
        scratch_types=[
            pltpu.VMEM((NPT, 5), _f32),
            pltpu.VMEM((ARPT, LANES), _i32),
            pltpu.VMEM((ARPT, LANES), _i32),
            pltpu.VMEM((DIB, LANES), _i32),
            pltpu.VMEM((LANES, CW), _f32),
            pltpu.VMEM((AIB, LANES, CW), _f32),
            pltpu.VMEM((AIB, LANES, CW), _f32),
            pltpu.VMEM_SHARED((N_ACC, CW), _f32),
            pltpu.SemaphoreType.DMA,
            pltpu.SemaphoreType.DMA,
        ],
    )(xp, dst2, tcat_c, z2d, ones2d)


def _sc_edges_body(y_hbm, src_hbm, dst_hbm, z2d_hbm, s_hbm,
                   sbuf0, sbuf1, dbuf0, dbuf1, gbuf0, gbuf1,
                   acc, sem_g, sem_s, sem_i):
    c = lax.axis_index("c")
    s = lax.axis_index("s")
    nbase = s * NPT
    row0 = s * RPT

    if True:
        for fc in range(CPC):
            f = CPC * c + fc
            pltpu.sync_copy(z2d_hbm, acc.at[pl.ds(s * NACC_T, NACC_T)])
            plsc.subcore_barrier()

            pltpu.async_copy(src_hbm.at[pl.ds(row0, EIB)], sbuf0, sem_i)
            pltpu.async_copy(dst_hbm.at[pl.ds(row0, EIB)], dbuf0, sem_i)

            def edge_pair(k, f=f):
                for par, (sb, db, gb, sbn, dbn) in (
                        (0, (sbuf0, dbuf0, gbuf0, sbuf1, dbuf1)),
                        (1, (sbuf1, dbuf1, gbuf1, sbuf0, dbuf0))):
                    b = 2 * k + par
                    pltpu.make_async_copy(src_hbm.at[pl.ds(0, EIB)], sb,
                                          sem_i).wait()
                    pltpu.make_async_copy(dst_hbm.at[pl.ds(0, EIB)], db,
                                          sem_i).wait()

                    @pl.when(k >= 1)
                    def _drain():
                        for j in range(EIB):
                            pltpu.make_async_copy(
                                y_hbm.at[0, pl.ds(0, LANES)], gb.at[j],
                                sem_s).wait()

                    if par == 0:
                        nxt = row0 + (b + 1) * EIB
                        pltpu.async_copy(src_hbm.at[pl.ds(nxt, EIB)], sbn,
                                         sem_i)
                        pltpu.async_copy(dst_hbm.at[pl.ds(nxt, EIB)], dbn,
                                         sem_i)
                    else:
                        @pl.when(k + 1 < EPAIRS)
                        def _pref():
                            nxt = row0 + (b + 1) * EIB
                            pltpu.async_copy(src_hbm.at[pl.ds(nxt, EIB)], sbn,
                                             sem_i)
                            pltpu.async_copy(dst_hbm.at[pl.ds(nxt, EIB)], dbn,
                                             sem_i)

                    descs = []
                    for j in range(EIB):
                        descs.append(pltpu.async_copy(
                            y_hbm.at[f].at[sb.at[j]], gb.at[j], sem_g))
                    for d in descs:
                        d.wait()
                    for j in range(EIB):
                        pltpu.async_copy(gb.at[j], acc.at[db.at[j]],
                                         sem_s, add=True)

            pl.loop(0, EPAIRS)(edge_pair)
            for gb in (gbuf0, gbuf1):
                for j in range(EIB):
                    pltpu.make_async_copy(
                        y_hbm.at[0, pl.ds(0, LANES)], gb.at[j], sem_s).wait()
            plsc.subcore_barrier()
            pltpu.sync_copy(acc.at[pl.ds(s * NACC_T, NACC_T)],
                            s_hbm.at[f, pl.ds(s * NACC_T, NACC_T)])
            plsc.subcore_barrier()


def _sc_edges(y_c, src2, dst2, z2d):
    mesh = plsc.VectorSubcoreMesh(core_axis_name="c", subcore_axis_name="s")
    return pl.kernel(
        _sc_edges_body,
        out_type=jax.ShapeDtypeStruct((CH, N_PAD, CW), _f32),
        mesh=mesh,
        compiler_params=pltpu.CompilerParams(use_tc_tiling_on_sc=False),
        scratch_types=[
            pltpu.VMEM((EIB, LANES), _i32),
            pltpu.VMEM((EIB, LANES), _i32),
            pltpu.VMEM((EIB, LANES), _i32),
            pltpu.VMEM((EIB, LANES), _i32),
            pltpu.VMEM((EIB, LANES, CW), _f32),
            pltpu.VMEM((EIB, LANES, CW), _f32),
            pltpu.VMEM_SHARED((N_ACC, CW), _f32),
            pltpu.SemaphoreType.DMA,
            pltpu.SemaphoreType.DMA,
            pltpu.SemaphoreType.DMA,
        ],
    )(y_c, src2, dst2, z2d)


def _tc_prep_body(embbd_ref, w1e_ref, out_ref):
    t = jnp.dot(embbd_ref[...], w1e_ref[...], preferred_element_type=_f32)
    for f in range(CH):
        out_ref[f] = t[:, CW * f:CW * (f + 1)]


def _tc_prep(embbd, w1e):
    return pl.pallas_call(
        _tc_prep_body,
        out_shape=jax.ShapeDtypeStruct((CH, TROWS, CW), _f32),
    )(embbd, w1e)


def _tc_k1_body(xw1_ref, degp_ref, poss_ref, w1r0_ref, y1_ref, dinv_ref):
    deg = degp_ref[0][:, 0:1] + degp_ref[1][:, 0:1] + 1.0
    dcol = 1.0 / jnp.sqrt(deg)
    dinv_ref[0] = dcol.reshape(1, 512)
    pcol = poss_ref[...].reshape(512, 1)
    pw = pcol * w1r0_ref[...]
    xw = jnp.concatenate([xw1_ref[f] for f in range(CH)], axis=-1)
    y = (xw + pw) * dcol
    for f in range(CH):
        y1_ref[f] = y[:, CW * f:CW * (f + 1)]


def _tc_k1(xw1_c, degp3, poss2, w1r0):
    return pl.pallas_call(
        _tc_k1_body,
        grid=(NBLK,),
        in_specs=[
            pl.BlockSpec((CH, 512, CW), lambda i: (0, i, 0)),
            pl.BlockSpec((NC, 512, CW), lambda i: (0, i, 0)),
            pl.BlockSpec((1, 1, 512), lambda i: (i, 0, 0)),
            pl.BlockSpec((1, HID), lambda i: (0, 0)),
        ],
        out_specs=[
            pl.BlockSpec((CH, 512, CW), lambda i: (0, i, 0)),
            pl.BlockSpec((1, 1, 512), lambda i: (i, 0, 0)),
        ],
        out_shape=[
            jax.ShapeDtypeStruct((CH, N_PAD, CW), _f32),
            jax.ShapeDtypeStruct((NBLK, 1, 512), _f32),
        ],
    )(xw1_c, degp3, poss2, w1r0)


def _tc_mid_body(s_ref, y_ref, dinv_ref, b_ref, g_ref, be_ref, w_ref,
                 out_ref, acc_s, acc_q):
    p = pl.program_id(0)
    i = pl.program_id(1)

    @pl.when(jnp.logical_and(p == 0, i == 0))
    def _init():
        acc_s[...] = jnp.zeros((1, HID), _f32)
        acc_q[...] = jnp.zeros((1, HID), _f32)

    dcol = dinv_ref[...].reshape(512, 1)
    zs = jnp.concatenate([s_ref[f] + y_ref[f] for f in range(CH)], axis=-1)
    z = dcol * zs + b_ref[...]

    @pl.when(p == 0)
    def _stats():
        gid = i * 512 + lax.broadcasted_iota(_i32, (512, 1), 0)
        m = gid < N
        zm = jnp.where(m, z, 0.0)
        zq = jnp.where(m, z * z, 0.0)
        acc_s[...] += jnp.sum(zm, axis=0, keepdims=True)
        acc_q[...] += jnp.sum(zq, axis=0, keepdims=True)

    @pl.when(p == 1)
    def _apply():
        mean = acc_s[...] / N
        var = acc_q[...] / N - mean * mean
        a = g_ref[...] / jnp.sqrt(var + 1e-5)
        cc = be_ref[...] - mean * a
        h = jnp.maximum(z * a + cc, 0.0)
        yn = jnp.dot(h, w_ref[...], preferred_element_type=_f32) * dcol
        for f in range(CH):
            out_ref[f] = yn[:, CW * f:CW * (f + 1)]


def _tc_mid(s_c, y_c, dinv2, b, g, be, w):
    return pl.pallas_call(
        _tc_mid_body,
        grid=(2, NBLK),
        in_specs=[
            pl.BlockSpec((CH, 512, CW), lambda p, i: (0, i, 0)),
            pl.BlockSpec((CH, 512, CW), lambda p, i: (0, i, 0)),
            pl.BlockSpec((1, 1, 512), lambda p, i: (i, 0, 0)),
            pl.BlockSpec((1, HID), lambda p, i: (0, 0)),
            pl.BlockSpec((1, HID), lambda p, i: (0, 0)),
            pl.BlockSpec((1, HID), lambda p, i: (0, 0)),
            pl.BlockSpec((HID, HID), lambda p, i: (0, 0)),
        ],
        out_specs=pl.BlockSpec((CH, 512, CW), lambda p, i: (0, i, 0)),
        out_shape=jax.ShapeDtypeStruct((CH, N_PAD, CW), _f32),
        scratch_shapes=[pltpu.VMEM((1, HID), _f32), pltpu.VMEM((1, HID), _f32)],
    )(s_c, y_c, dinv2, b, g, be, w)


def _tc_final_body(s_ref, y_ref, dinv_ref, b_ref, g_ref, be_ref,
                   batch_ref, wfc_ref, bfc_ref,
                   out_ref, acc_s, acc_q, pooled, cnt):
    p = pl.program_id(0)
    i = pl.program_id(1)

    @pl.when(jnp.logical_and(p == 0, i == 0))
    def _init():
        acc_s[...] = jnp.zeros((1, HID), _f32)
        acc_q[...] = jnp.zeros((1, HID), _f32)
        pooled[...] = jnp.zeros((G, HID), _f32)
        cnt[...] = jnp.zeros((1, G), _f32)

    dcol = dinv_ref[...].reshape(512, 1)
    zs = jnp.concatenate([s_ref[f] + y_ref[f] for f in range(CH)], axis=-1)
    z = dcol * zs + b_ref[...]

    @pl.when(p == 0)
    def _stats():
        gid = i * 512 + lax.broadcasted_iota(_i32, (512, 1), 0)
        m = gid < N
        zm = jnp.where(m, z, 0.0)
        zq = jnp.where(m, z * z, 0.0)
        acc_s[...] += jnp.sum(zm, axis=0, keepdims=True)
        acc_q[...] += jnp.sum(zq, axis=0, keepdims=True)

    @pl.when(p == 1)
    def _pool():
        mean = acc_s[...] / N
        var = acc_q[...] / N - mean * mean
        a = g_ref[...] / jnp.sqrt(var + 1e-5)
        cc = be_ref[...] - mean * a
        gid = i * 512 + lax.broadcasted_iota(_i32, (512, 1), 0)
        h = jnp.where(gid < N, jnp.maximum(z * a + cc, 0.0), 0.0)
        bcol = batch_ref[...].reshape(512, 1)
        segs = lax.broadcasted_iota(_i32, (512, G), 1)
        oh = (segs == bcol).astype(_f32)
        pooled[...] += lax.dot_general(oh, h, (((0,), (0,)), ((), ())),
                                       preferred_element_type=_f32)
        cnt[...] += jnp.sum(oh, axis=0, keepdims=True)

    @pl.when(jnp.logical_and(p == 1, i == NBLK - 1))
    def _fc():
        ccol = cnt[...].reshape(G, 1)
        meanp = pooled[...] / jnp.maximum(ccol, 1.0)
        out_ref[...] = (jnp.dot(meanp, wfc_ref[...],
                                preferred_element_type=_f32) + bfc_ref[...])


def _tc_final(s_c, y_c, dinv2, b, g, be, batch2, wfc_p, bfc_p):
    return pl.pallas_call(
        _tc_final_body,
        grid=(2, NBLK),
        in_specs=[
            pl.BlockSpec((CH, 512, CW), lambda p, i: (0, i, 0)),
            pl.BlockSpec((CH, 512, CW), lambda p, i: (0, i, 0)),
            pl.BlockSpec((1, 1, 512), lambda p, i: (i, 0, 0)),
            pl.BlockSpec((1, HID), lambda p, i: (0, 0)),
            pl.BlockSpec((1, HID), lambda p, i: (0, 0)),
            pl.BlockSpec((1, HID), lambda p, i: (0, 0)),
            pl.BlockSpec((1, 1, 512), lambda p, i: (i, 0, 0)),
            pl.BlockSpec((HID, HID), lambda p, i: (0, 0)),
            pl.BlockSpec((1, HID), lambda p, i: (0, 0)),
        ],
        out_specs=pl.BlockSpec((G, HID), lambda p, i: (0, 0)),
        out_shape=jax.ShapeDtypeStruct((G, HID), _f32),
        scratch_shapes=[
            pltpu.VMEM((1, HID), _f32),
            pltpu.VMEM((1, HID), _f32),
            pltpu.VMEM((G, HID), _f32),
            pltpu.VMEM((1, G), _f32),
        ],
    )(s_c, y_c, dinv2, b, g, be, batch2, wfc_p, bfc_p)


def kernel(x, edge_index, batch, emb0, emb1, emb2, emb5,
           W1, b1, W2, b2, W3, b3, g1, be1, g2, be2, g3, be3, Wfc, bfc):
    xp = jnp.pad(x, ((0, N_PAD - N), (0, 0)))
    src = edge_index[0]
    dst = edge_index[1]
    src2 = jnp.concatenate([src, jnp.zeros((E_PAD - E,), _i32)]
                           ).reshape(E_ROWS, LANES)
    dst2 = jnp.concatenate([dst, jnp.full((E_PAD - E,), N, _i32)]
                           ).reshape(E_ROWS, LANES)
    batch2 = jnp.concatenate([batch, jnp.full((N_PAD - N,), G, _i32)]
                             ).reshape(NBLK, 1, 512)
    poss2 = xp[:, 0].reshape(NBLK, 1, 512)
    z2d = jnp.zeros((NACC_T, CW), _f32)
    ones2d = jnp.ones((LANES, CW), _f32)

    embbd = jnp.zeros((TROWS, HID), _f32)
    embbd = embbd.at[0:96, 0:32].set(emb0)
    embbd = embbd.at[96:192, 32:64].set(emb1)
    embbd = embbd.at[192:288, 64:96].set(emb2)
    embbd = embbd.at[288:288 + 182, 96:128].set(emb5)
    w1e = W1[1:129]
    w1r0 = W1[0:1]
    wfc_p = jnp.pad(Wfc, ((0, 0), (0, HID - Wfc.shape[1])))
    bfc_p = jnp.pad(bfc, (0, HID - bfc.shape[0])).reshape(1, HID)

    tcat_c = _tc_prep(embbd, w1e)
    xw1_c, degp = _sc_emb_deg(xp, dst2, tcat_c, z2d, ones2d)
    y1_c, dinv2 = _tc_k1(xw1_c, degp, poss2, w1r0)

    s1_c = _sc_edges(y1_c, src2, dst2, z2d)
    y2_c = _tc_mid(s1_c, y1_c, dinv2, b1.reshape(1, HID), g1.reshape(1, HID),
                   be1.reshape(1, HID), W2)
    s2_c = _sc_edges(y2_c, src2, dst2, z2d)
    y3_c = _tc_mid(s2_c, y2_c, dinv2, b2.reshape(1, HID), g2.reshape(1, HID),
                   be2.reshape(1, HID), W3)
    s3_c = _sc_edges(y3_c, src2, dst2, z2d)
    outp = _tc_final(s3_c, y3_c, dinv2, b3.reshape(1, HID), g3.reshape(1, HID),
                     be3.reshape(1, HID), batch2, wfc_p, bfc_p)
    return outp[:, :3]


# speedup vs baseline: 3.4611x; 1.0711x over previous
import jax
import jax.numpy as jnp
from jax import lax
from jax.experimental import pallas as pl
from jax.experimental.pallas import tpu as pltpu
from jax.experimental.pallas import tpu_sc as plsc

N = 50000
E = 800000
G = 512
HID = 128

NC = 2
NS = 16
LANES = 128

N_PAD = 50176
NPT = N_PAD // NS
E_PAD = 819200
EPT = E_PAD // NS
EBLK = 1280
ENB = EPT // EBLK
EPAIRS = ENB // 2
DPT = E_PAD // NC // NS
DNB = DPT // EBLK
DPAIRS = DNB // 2
ARPT = NPT * 4 // LANES
AIB = 7
ABLK = AIB * LANES
APAIRS = ARPT // (2 * AIB)
NBLK = N_PAD // 512
CH = 16
CW = 8
CPC = CH // NC
N_ACC = 50016
NACC_T = N_ACC // NS
TROWS = 480

_f32 = jnp.float32
_i32 = jnp.int32


def _sc_emb_deg_body(x_hbm, dst_hbm, tcat_hbm, z2d_hbm, ones_hbm,
                     xw1_hbm, degp_hbm,
                     xbuf, gidx, nidx, dbuf0, dbuf1, obuf, gbuf0, gbuf1,
                     acc, sem_g, sem_s):
    c = lax.axis_index("c")
    s = lax.axis_index("s")
    nbase = s * NPT

    pltpu.sync_copy(ones_hbm, obuf)

    pltpu.sync_copy(x_hbm.at[pl.ds(nbase, NPT)], xbuf)
    lane = lax.iota(_i32, 16)

    def build_row(r):
        for half in range(2):
            rows16 = (2 * r + half) * 16 + lane
            nid = jnp.minimum(nbase + rows16, N)
            for t, (col, off) in enumerate(((1, 0), (2, 96), (3, 192), (4, 288))):
                vals = plsc.load_gather(xbuf, [rows16, jnp.full((16,), col, _i32)])
                iv = vals.astype(_i32) + off
                gidx[pl.ds(r * LANES + 32 * t + 16 * half, 16)] = iv
                nidx[pl.ds(r * LANES + 32 * t + 16 * half, 16)] = nid

    pl.loop(0, ARPT)(build_row)

    pltpu.sync_copy(z2d_hbm, acc.at[pl.ds(s * NACC_T, NACC_T)])
    plsc.subcore_barrier()
    dbase = (c * NS + s) * DPT

    def deg_pair(k):
        for par, db in ((0, dbuf0), (1, dbuf1)):
            b = 2 * k + par

            @pl.when(k >= 1)
            def _drain():
                pltpu.make_async_copy(ones_hbm, obuf, sem_s).wait()

            pltpu.sync_copy(dst_hbm.at[pl.ds(dbase + b * EBLK, EBLK)], db)
            pltpu.async_copy(obuf, acc.at[db], sem_s, add=True)

    pl.loop(0, DPAIRS)(deg_pair)
    for _ in range(2):
        pltpu.make_async_copy(ones_hbm, obuf, sem_s).wait()
    plsc.subcore_barrier()
    pltpu.sync_copy(acc.at[pl.ds(s * NACC_T, NACC_T)],
                    degp_hbm.at[c, pl.ds(s * NACC_T, NACC_T)])
    plsc.subcore_barrier()

    for fc in range(CPC):
        f = CPC * c + fc
        pltpu.sync_copy(z2d_hbm, acc.at[pl.ds(s * NACC_T, NACC_T)])
        plsc.subcore_barrier()

        def chunk_pair(k, f=f):
            for par, gb in ((0, gbuf0), (1, gbuf1)):
                b = 2 * k + par

                @pl.when(k >= 1)
                def _drain():
                    pltpu.make_async_copy(
                        xw1_hbm.at[0, pl.ds(0, ABLK)], gb, sem_s).wait()

                pltpu.async_copy(
                    tcat_hbm.at[f].at[gidx.at[pl.ds(b * ABLK, ABLK)]],
                    gb, sem_g).wait()
                pltpu.async_copy(gb, acc.at[nidx.at[pl.ds(b * ABLK, ABLK)]],
                                 sem_s, add=True)

        pl.loop(0, APAIRS)(chunk_pair)
        for gb in (gbuf0, gbuf1):
            pltpu.make_async_copy(
                xw1_hbm.at[0, pl.ds(0, ABLK)], gb, sem_s).wait()
        plsc.subcore_barrier()
        pltpu.sync_copy(acc.at[pl.ds(s * NACC_T, NACC_T)],
                        xw1_hbm.at[f, pl.ds(s * NACC_T, NACC_T)])
        plsc.subcore_barrier()


def _sc_emb_deg(xp, dst1, tcat_c, z2d, ones2d):
    mesh = plsc.VectorSubcoreMesh(core_axis_name="c", subcore_axis_name="s")
    return pl.kernel(
        _sc_emb_deg_body,
        out_type=(jax.ShapeDtypeStruct((CH, N_PAD, CW), _f32),
                  jax.ShapeDtypeStruct((NC, N_PAD, CW), _f32)),
        mesh=mesh,
        compiler_params=pltpu.CompilerParams(needs_layout_passes=False,
                                             use_tc_tiling_on_sc=False),
        scratch_types=[
            pltpu.VMEM((NPT, 5), _f32),
            pltpu.VMEM((ARPT * LANES,), _i32),
            pltpu.VMEM((ARPT * LANES,), _i32),
            pltpu.VMEM((EBLK,), _i32),
            pltpu.VMEM((EBLK,), _i32),
            pltpu.VMEM((EBLK, CW), _f32),
            pltpu.VMEM((ABLK, CW), _f32),
            pltpu.VMEM((ABLK, CW), _f32),
            pltpu.VMEM_SHARED((N_ACC, CW), _f32),
            pltpu.SemaphoreType.DMA,
            pltpu.SemaphoreType.DMA,
        ],
    )(xp, dst1, tcat_c, z2d, ones2d)


def _sc_edges_body(y_hbm, src_hbm, dst_hbm, z2d_hbm, s_hbm,
                   sbuf0, sbuf1, dbuf0, dbuf1, gbuf0, gbuf1,
                   acc, sem_g, sem_s, sem_i):
    c = lax.axis_index("c")
    s = lax.axis_index("s")
    ebase = s * EPT

    for fc in range(CPC):
        f = CPC * c + fc
        pltpu.sync_copy(z2d_hbm, acc.at[pl.ds(s * NACC_T, NACC_T)])
        plsc.subcore_barrier()

        pltpu.async_copy(src_hbm.at[pl.ds(ebase, EBLK)], sbuf0, sem_i)
        pltpu.async_copy(dst_hbm.at[pl.ds(ebase, EBLK)], dbuf0, sem_i)

        def edge_pair(k, f=f):
            for par, (sb, db, gb, sbn, dbn) in (
                    (0, (sbuf0, dbuf0, gbuf0, sbuf1, dbuf1)),
                    (1, (sbuf1, dbuf1, gbuf1, sbuf0, dbuf0))):
                b = 2 * k + par
                pltpu.make_async_copy(src_hbm.at[pl.ds(0, EBLK)], sb,
                                      sem_i).wait()
                pltpu.make_async_copy(dst_hbm.at[pl.ds(0, EBLK)], db,
                                      sem_i).wait()

                @pl.when(k >= 1)
                def _drain():
                    pltpu.make_async_copy(
                        y_hbm.at[0, pl.ds(0, EBLK)], gb, sem_s).wait()

                if par == 0:
                    nxt = ebase + (b + 1) * EBLK
                    pltpu.async_copy(src_hbm.at[pl.ds(nxt, EBLK)], sbn, sem_i)
                    pltpu.async_copy(dst_hbm.at[pl.ds(nxt, EBLK)], dbn, sem_i)
                else:
                    @pl.when(k + 1 < EPAIRS)
                    def _pref():
                        nxt = ebase + (b + 1) * EBLK
                        pltpu.async_copy(src_hbm.at[pl.ds(nxt, EBLK)], sbn,
                                         sem_i)
                        pltpu.async_copy(dst_hbm.at[pl.ds(nxt, EBLK)], dbn,
                                         sem_i)

                pltpu.async_copy(y_hbm.at[f].at[sb], gb, sem_g).wait()
                pltpu.async_copy(gb, acc.at[db], sem_s, add=True)

        pl.loop(0, EPAIRS)(edge_pair)
        for gb in (gbuf0, gbuf1):
            pltpu.make_async_copy(
                y_hbm.at[0, pl.ds(0, EBLK)], gb, sem_s).wait()
        plsc.subcore_barrier()
        pltpu.sync_copy(acc.at[pl.ds(s * NACC_T, NACC_T)],
                        s_hbm.at[f, pl.ds(s * NACC_T, NACC_T)])
        plsc.subcore_barrier()


def _sc_edges(y_c, src1, dst1, z2d):
    mesh = plsc.VectorSubcoreMesh(core_axis_name="c", subcore_axis_name="s")
    return pl.kernel(
        _sc_edges_body,
        out_type=jax.ShapeDtypeStruct((CH, N_PAD, CW), _f32),
        mesh=mesh,
        compiler_params=pltpu.CompilerParams(use_tc_tiling_on_sc=False),
        scratch_types=[
            pltpu.VMEM((EBLK,), _i32),
            pltpu.VMEM((EBLK,), _i32),
            pltpu.VMEM((EBLK,), _i32),
            pltpu.VMEM((EBLK,), _i32),
            pltpu.VMEM((EBLK, CW), _f32),
            pltpu.VMEM((EBLK, CW), _f32),
            pltpu.VMEM_SHARED((N_ACC, CW), _f32),
            pltpu.SemaphoreType.DMA,
            pltpu.SemaphoreType.DMA,
            pltpu.SemaphoreType.DMA,
        ],
    )(y_c, src1, dst1, z2d)


def _tc_prep_body(embbd_ref, w1e_ref, out_ref):
    t = jnp.dot(embbd_ref[...], w1e_ref[...], preferred_element_type=_f32)
    for f in range(CH):
        out_ref[f] = t[:, CW * f:CW * (f + 1)]


def _tc_prep(embbd, w1e):
    return pl.pallas_call(
        _tc_prep_body,
        out_shape=jax.ShapeDtypeStruct((CH, TROWS, CW), _f32),
    )(embbd, w1e)


def _tc_k1_body(xw1_ref, degp_ref, poss_ref, w1r0_ref, y1_ref, dinv_ref):
    deg = degp_ref[0][:, 0:1] + degp_ref[1][:, 0:1] + 1.0
    dcol = 1.0 / jnp.sqrt(deg)
    dinv_ref[0] = dcol.reshape(1, 512)
    pcol = poss_ref[...].reshape(512, 1)
    pw = pcol * w1r0_ref[...]
    xw = jnp.concatenate([xw1_ref[f] for f in range(CH)], axis=-1)
    y = (xw + pw) * dcol
    for f in range(CH):
        y1_ref[f] = y[:, CW * f:CW * (f + 1)]


def _tc_k1(xw1_c, degp3, poss2, w1r0):
    return pl.pallas_call(
        _tc_k1_body,
        grid=(NBLK,),
        in_specs=[
            pl.BlockSpec((CH, 512, CW), lambda i: (0, i, 0)),
            pl.BlockSpec((NC, 512, CW), lambda i: (0, i, 0)),
            pl.BlockSpec((1, 1, 512), lambda i: (i, 0, 0)),
            pl.BlockSpec((1, HID), lambda i: (0, 0)),
        ],
        out_specs=[
            pl.BlockSpec((CH, 512, CW), lambda i: (0, i, 0)),
            pl.BlockSpec((1, 1, 512), lambda i: (i, 0, 0)),
        ],
        out_shape=[
            jax.ShapeDtypeStruct((CH, N_PAD, CW), _f32),
            jax.ShapeDtypeStruct((NBLK, 1, 512), _f32),
        ],
    )(xw1_c, degp3, poss2, w1r0)


def _tc_mid_body(s_ref, y_ref, dinv_ref, b_ref, g_ref, be_ref, w_ref,
                 out_ref, acc_s, acc_q):
    p = pl.program_id(0)
    i = pl.program_id(1)

    @pl.when(jnp.logical_and(p == 0, i == 0))
    def _init():
        acc_s[...] = jnp.zeros((1, HID), _f32)
        acc_q[...] = jnp.zeros((1, HID), _f32)

    dcol = dinv_ref[...].reshape(512, 1)
    zs = jnp.concatenate([s_ref[f] + y_ref[f] for f in range(CH)], axis=-1)
    z = dcol * zs + b_ref[...]

    @pl.when(p == 0)
    def _stats():
        gid = i * 512 + lax.broadcasted_iota(_i32, (512, 1), 0)
        m = gid < N
        zm = jnp.where(m, z, 0.0)
        zq = jnp.where(m, z * z, 0.0)
        acc_s[...] += jnp.sum(zm, axis=0, keepdims=True)
        acc_q[...] += jnp.sum(zq, axis=0, keepdims=True)

    @pl.when(p == 1)
    def _apply():
        mean = acc_s[...] / N
        var = acc_q[...] / N - mean * mean
        a = g_ref[...] / jnp.sqrt(var + 1e-5)
        cc = be_ref[...] - mean * a
        h = jnp.maximum(z * a + cc, 0.0)
        yn = jnp.dot(h, w_ref[...], preferred_element_type=_f32) * dcol
        for f in range(CH):
            out_ref[f] = yn[:, CW * f:CW * (f + 1)]


def _tc_mid(s_c, y_c, dinv2, b, g, be, w):
    return pl.pallas_call(
        _tc_mid_body,
        grid=(2, NBLK),
        in_specs=[
            pl.BlockSpec((CH, 512, CW), lambda p, i: (0, i, 0)),
            pl.BlockSpec((CH, 512, CW), lambda p, i: (0, i, 0)),
            pl.BlockSpec((1, 1, 512), lambda p, i: (i, 0, 0)),
            pl.BlockSpec((1, HID), lambda p, i: (0, 0)),
            pl.BlockSpec((1, HID), lambda p, i: (0, 0)),
            pl.BlockSpec((1, HID), lambda p, i: (0, 0)),
            pl.BlockSpec((HID, HID), lambda p, i: (0, 0)),
        ],
        out_specs=pl.BlockSpec((CH, 512, CW), lambda p, i: (0, i, 0)),
        out_shape=jax.ShapeDtypeStruct((CH, N_PAD, CW), _f32),
        scratch_shapes=[pltpu.VMEM((1, HID), _f32), pltpu.VMEM((1, HID), _f32)],
    )(s_c, y_c, dinv2, b, g, be, w)


def _tc_final_body(s_ref, y_ref, dinv_ref, b_ref, g_ref, be_ref,
                   batch_ref, wfc_ref, bfc_ref,
                   out_ref, acc_s, acc_q, pooled, cnt):
    p = pl.program_id(0)
    i = pl.program_id(1)

    @pl.when(jnp.logical_and(p == 0, i == 0))
    def _init():
        acc_s[...] = jnp.zeros((1, HID), _f32)
        acc_q[...] = jnp.zeros((1, HID), _f32)
        pooled[...] = jnp.zeros((G, HID), _f32)
        cnt[...] = jnp.zeros((1, G), _f32)

    dcol = dinv_ref[...].reshape(512, 1)
    zs = jnp.concatenate([s_ref[f] + y_ref[f] for f in range(CH)], axis=-1)
    z = dcol * zs + b_ref[...]

    @pl.when(p == 0)
    def _stats():
        gid = i * 512 + lax.broadcasted_iota(_i32, (512, 1), 0)
        m = gid < N
        zm = jnp.where(m, z, 0.0)
        zq = jnp.where(m, z * z, 0.0)
        acc_s[...] += jnp.sum(zm, axis=0, keepdims=True)
        acc_q[...] += jnp.sum(zq, axis=0, keepdims=True)

    @pl.when(p == 1)
    def _pool():
        mean = acc_s[...] / N
        var = acc_q[...] / N - mean * mean
        a = g_ref[...] / jnp.sqrt(var + 1e-5)
        cc = be_ref[...] - mean * a
        gid = i * 512 + lax.broadcasted_iota(_i32, (512, 1), 0)
        h = jnp.where(gid < N, jnp.maximum(z * a + cc, 0.0), 0.0)
        bcol = batch_ref[...].reshape(512, 1)
        segs = lax.broadcasted_iota(_i32, (512, G), 1)
        oh = (segs == bcol).astype(_f32)
        pooled[...] += lax.dot_general(oh, h, (((0,), (0,)), ((), ())),
                                       preferred_element_type=_f32)
        cnt[...] += jnp.sum(oh, axis=0, keepdims=True)

    @pl.when(jnp.logical_and(p == 1, i == NBLK - 1))
    def _fc():
        ccol = cnt[...].reshape(G, 1)
        meanp = pooled[...] / jnp.maximum(ccol, 1.0)
        out_ref[...] = (jnp.dot(meanp, wfc_ref[...],
                                preferred_element_type=_f32) + bfc_ref[...])


def _tc_final(s_c, y_c, dinv2, b, g, be, batch2, wfc_p, bfc_p):
    return pl.pallas_call(
        _tc_final_body,
        grid=(2, NBLK),
        in_specs=[
            pl.BlockSpec((CH, 512, CW), lambda p, i: (0, i, 0)),
            pl.BlockSpec((CH, 512, CW), lambda p, i: (0, i, 0)),
            pl.BlockSpec((1, 1, 512), lambda p, i: (i, 0, 0)),
            pl.BlockSpec((1, HID), lambda p, i: (0, 0)),
            pl.BlockSpec((1, HID), lambda p, i: (0, 0)),
            pl.BlockSpec((1, HID), lambda p, i: (0, 0)),
            pl.BlockSpec((1, 1, 512), lambda p, i: (i, 0, 0)),
            pl.BlockSpec((HID, HID), lambda p, i: (0, 0)),
            pl.BlockSpec((1, HID), lambda p, i: (0, 0)),
        ],
        out_specs=pl.BlockSpec((G, HID), lambda p, i: (0, 0)),
        out_shape=jax.ShapeDtypeStruct((G, HID), _f32),
        scratch_shapes=[
            pltpu.VMEM((1, HID), _f32),
            pltpu.VMEM((1, HID), _f32),
            pltpu.VMEM((G, HID), _f32),
            pltpu.VMEM((1, G), _f32),
        ],
    )(s_c, y_c, dinv2, b, g, be, batch2, wfc_p, bfc_p)


def kernel(x, edge_index, batch, emb0, emb1, emb2, emb5,
           W1, b1, W2, b2, W3, b3, g1, be1, g2, be2, g3, be3, Wfc, bfc):
    xp = jnp.pad(x, ((0, N_PAD - N), (0, 0)))
    src = edge_index[0]
    dst = edge_index[1]
    src1 = jnp.concatenate([src, jnp.zeros((E_PAD - E,), _i32)])
    dst1 = jnp.concatenate([dst, jnp.full((E_PAD - E,), N, _i32)])
    batch2 = jnp.concatenate([batch, jnp.full((N_PAD - N,), G, _i32)]
                             ).reshape(NBLK, 1, 512)
    poss2 = xp[:, 0].reshape(NBLK, 1, 512)
    z2d = jnp.zeros((NACC_T, CW), _f32)
    ones2d = jnp.ones((EBLK, CW), _f32)

    embbd = jnp.zeros((TROWS, HID), _f32)
    embbd = embbd.at[0:96, 0:32].set(emb0)
    embbd = embbd.at[96:192, 32:64].set(emb1)
    embbd = embbd.at[192:288, 64:96].set(emb2)
    embbd = embbd.at[288:288 + 182, 96:128].set(emb5)
    w1e = W1[1:129]
    w1r0 = W1[0:1]
    wfc_p = jnp.pad(Wfc, ((0, 0), (0, HID - Wfc.shape[1])))
    bfc_p = jnp.pad(bfc, (0, HID - bfc.shape[0])).reshape(1, HID)

    tcat_c = _tc_prep(embbd, w1e)
    xw1_c, degp = _sc_emb_deg(xp, dst1, tcat_c, z2d, ones2d)
    y1_c, dinv2 = _tc_k1(xw1_c, degp, poss2, w1r0)

    s1_c = _sc_edges(y1_c, src1, dst1, z2d)
    y2_c = _tc_mid(s1_c, y1_c, dinv2, b1.reshape(1, HID), g1.reshape(1, HID),
                   be1.reshape(1, HID), W2)
    s2_c = _sc_edges(y2_c, src1, dst1, z2d)
    y3_c = _tc_mid(s2_c, y2_c, dinv2, b2.reshape(1, HID), g2.reshape(1, HID),
                   be2.reshape(1, HID), W3)
    s3_c = _sc_edges(y3_c, src1, dst1, z2d)
    outp = _tc_final(s3_c, y3_c, dinv2, b3.reshape(1, HID), g3.reshape(1, HID),
                     be3.reshape(1, HID), batch2, wfc_p, bfc_p)
    return outp[:, :3]


# speedup vs baseline: 4.3639x; 1.2608x over previous
import jax
import jax.numpy as jnp
from jax import lax
from jax.experimental import pallas as pl
from jax.experimental.pallas import tpu as pltpu
from jax.experimental.pallas import tpu_sc as plsc

N = 50000
E = 800000
G = 512
HID = 128

NC = 2
NS = 16
LANES = 128

N_PAD = 50176
NPT = N_PAD // NS
E_PAD = 860160
EPT = E_PAD // NS
EBLK = 1120
DPT = E_PAD // NC // NS
DPAIRS = DPT // EBLK // 2
ARPT = NPT * 4 // LANES
AIB = 7
ABLK = AIB * LANES
APAIRS = ARPT // (2 * AIB)
NBLK = N_PAD // 512
CH = 16
CW = 8
CPC = CH // NC
N_ACC = 50016
NACC_T = N_ACC // NS
TROWS = 480

_f32 = jnp.float32
_i32 = jnp.int32


def _sc_emb_deg_body(x_hbm, dst_hbm, tcat_hbm, z2d_hbm, ones_hbm,
                     xw1_hbm, degp_hbm,
                     xbuf, gidx, nidx, dbuf0, dbuf1, obuf, gbuf0, gbuf1,
                     acc, sem_g, sem_s):
    c = lax.axis_index("c")
    s = lax.axis_index("s")
    nbase = s * NPT

    pltpu.sync_copy(ones_hbm, obuf)

    pltpu.sync_copy(x_hbm.at[pl.ds(nbase, NPT)], xbuf)
    lane = lax.iota(_i32, 16)

    def build_row(r):
        for half in range(2):
            rows16 = (2 * r + half) * 16 + lane
            nid = jnp.minimum(nbase + rows16, N)
            for t, (col, off) in enumerate(((1, 0), (2, 96), (3, 192), (4, 288))):
                vals = plsc.load_gather(xbuf, [rows16, jnp.full((16,), col, _i32)])
                iv = vals.astype(_i32) + off
                gidx[pl.ds(r * LANES + 32 * t + 16 * half, 16)] = iv
                nidx[pl.ds(r * LANES + 32 * t + 16 * half, 16)] = nid

    pl.loop(0, ARPT)(build_row)

    pltpu.sync_copy(z2d_hbm, acc.at[pl.ds(s * NACC_T, NACC_T)])
    plsc.subcore_barrier()
    dbase = (c * NS + s) * DPT

    def deg_pair(k):
        for par, db in ((0, dbuf0), (1, dbuf1)):
            b = 2 * k + par

            @pl.when(k >= 1)
            def _drain():
                pltpu.make_async_copy(ones_hbm, obuf, sem_s).wait()

            pltpu.sync_copy(dst_hbm.at[pl.ds(dbase + b * EBLK, EBLK)], db)
            pltpu.async_copy(obuf, acc.at[db], sem_s, add=True)

    pl.loop(0, DPAIRS)(deg_pair)
    for _ in range(2):
        pltpu.make_async_copy(ones_hbm, obuf, sem_s).wait()
    plsc.subcore_barrier()
    pltpu.sync_copy(acc.at[pl.ds(s * NACC_T, NACC_T)],
                    degp_hbm.at[c, pl.ds(s * NACC_T, NACC_T)])

    for fc in range(CPC):
        f = CPC * c + fc
        pltpu.sync_copy(z2d_hbm, acc.at[pl.ds(s * NACC_T, NACC_T)])
        plsc.subcore_barrier()

        def chunk_pair(k, f=f):
            for par, gb in ((0, gbuf0), (1, gbuf1)):
                b = 2 * k + par

                @pl.when(k >= 1)
                def _drain():
                    pltpu.make_async_copy(
                        xw1_hbm.at[0, pl.ds(0, ABLK)], gb, sem_s).wait()

                pltpu.async_copy(
                    tcat_hbm.at[f].at[gidx.at[pl.ds(b * ABLK, ABLK)]],
                    gb, sem_g).wait()
                pltpu.async_copy(gb, acc.at[nidx.at[pl.ds(b * ABLK, ABLK)]],
                                 sem_s, add=True)

        pl.loop(0, APAIRS)(chunk_pair)
        for gb in (gbuf0, gbuf1):
            pltpu.make_async_copy(
                xw1_hbm.at[0, pl.ds(0, ABLK)], gb, sem_s).wait()
        plsc.subcore_barrier()
        pltpu.sync_copy(acc.at[pl.ds(s * NACC_T, NACC_T)],
                        xw1_hbm.at[f, pl.ds(s * NACC_T, NACC_T)])


def _sc_emb_deg(xp, dst1, tcat_c, z2d, ones2d):
    mesh = plsc.VectorSubcoreMesh(core_axis_name="c", subcore_axis_name="s")
    return pl.kernel(
        _sc_emb_deg_body,
        out_type=(jax.ShapeDtypeStruct((CH, N_PAD, CW), _f32),
                  jax.ShapeDtypeStruct((NC, N_PAD, CW), _f32)),
        mesh=mesh,
        compiler_params=pltpu.CompilerParams(needs_layout_passes=False,
                                             use_tc_tiling_on_sc=False),
        scratch_types=[
            pltpu.VMEM((NPT, 5), _f32),
            pltpu.VMEM((ARPT * LANES,), _i32),
            pltpu.VMEM((ARPT * LANES,), _i32),
            pltpu.VMEM((EBLK,), _i32),
            pltpu.VMEM((EBLK,), _i32),
            pltpu.VMEM((EBLK, CW), _f32),
            pltpu.VMEM((ABLK, CW), _f32),
            pltpu.VMEM((ABLK, CW), _f32),
            pltpu.VMEM_SHARED((N_ACC, CW), _f32),
            pltpu.SemaphoreType.DMA,
            pltpu.SemaphoreType.DMA,
        ],
    )(xp, dst1, tcat_c, z2d, ones2d)


EHALF = EPT // 2
EQ = EHALF // (4 * EBLK)


def _sc_edges_body(y_hbm, src_hbm, dst_hbm, z2d_hbm, s_hbm,
                   sidx, didx, gbuf0, gbuf1, gbuf2, gbuf3,
                   acc, sem_g, sem_s):
    c = lax.axis_index("c")
    s = lax.axis_index("s")
    ebase = s * EPT
    gbufs = (gbuf0, gbuf1, gbuf2, gbuf3)

    for fc in range(CPC):
        f = CPC * c + fc
        pltpu.sync_copy(z2d_hbm, acc.at[pl.ds(s * NACC_T, NACC_T)])
        plsc.subcore_barrier()

        for half in range(2):
            if half == 1:
                for gb in gbufs:
                    pltpu.make_async_copy(
                        y_hbm.at[0, pl.ds(0, EBLK)], gb, sem_s).wait()
            hb = ebase + half * EHALF
            pltpu.sync_copy(src_hbm.at[pl.ds(hb, EHALF)], sidx)
            pltpu.sync_copy(dst_hbm.at[pl.ds(hb, EHALF)], didx)

            def quad(q, f=f):
                descs = []
                for j in range(4):
                    gb = gbufs[j]

                    @pl.when(q >= 1)
                    def _drain(gb=gb):
                        pltpu.make_async_copy(
                            y_hbm.at[0, pl.ds(0, EBLK)], gb, sem_s).wait()

                    off = (4 * q + j) * EBLK
                    descs.append(pltpu.async_copy(
                        y_hbm.at[f].at[sidx.at[pl.ds(off, EBLK)]], gb, sem_g))
                for j in range(4):
                    descs[j].wait()
                    off = (4 * q + j) * EBLK
                    pltpu.async_copy(gbufs[j],
                                     acc.at[didx.at[pl.ds(off, EBLK)]],
                                     sem_s, add=True)

            pl.loop(0, EQ)(quad)

        for gb in gbufs:
            pltpu.make_async_copy(
                y_hbm.at[0, pl.ds(0, EBLK)], gb, sem_s).wait()
        plsc.subcore_barrier()
        pltpu.sync_copy(acc.at[pl.ds(s * NACC_T, NACC_T)],
                        s_hbm.at[f, pl.ds(s * NACC_T, NACC_T)])


def _sc_edges(y_c, src1, dst1, z2d):
    mesh = plsc.VectorSubcoreMesh(core_axis_name="c", subcore_axis_name="s")
    return pl.kernel(
        _sc_edges_body,
        out_type=jax.ShapeDtypeStruct((CH, N_PAD, CW), _f32),
        mesh=mesh,
        compiler_params=pltpu.CompilerParams(use_tc_tiling_on_sc=False),
        scratch_types=[
            pltpu.VMEM((EHALF,), _i32),
            pltpu.VMEM((EHALF,), _i32),
            pltpu.VMEM((EBLK, CW), _f32),
            pltpu.VMEM((EBLK, CW), _f32),
            pltpu.VMEM((EBLK, CW), _f32),
            pltpu.VMEM((EBLK, CW), _f32),
            pltpu.VMEM_SHARED((N_ACC, CW), _f32),
            pltpu.SemaphoreType.DMA,
            pltpu.SemaphoreType.DMA,
        ],
    )(y_c, src1, dst1, z2d)


def _tc_prep_body(embbd_ref, w1e_ref, out_ref):
    t = jnp.dot(embbd_ref[...], w1e_ref[...], preferred_element_type=_f32)
    for f in range(CH):
        out_ref[f] = t[:, CW * f:CW * (f + 1)]


def _tc_prep(embbd, w1e):
    return pl.pallas_call(
        _tc_prep_body,
        out_shape=jax.ShapeDtypeStruct((CH, TROWS, CW), _f32),
    )(embbd, w1e)


def _tc_k1_body(xw1_ref, degp_ref, poss_ref, w1r0_ref, y1_ref, dinv_ref):
    deg = degp_ref[0][:, 0:1] + degp_ref[1][:, 0:1]
    dcol = 1.0 / jnp.sqrt(deg)
    dinv_ref[0] = dcol.reshape(1, 512)
    pcol = poss_ref[...].reshape(512, 1)
    pw = pcol * w1r0_ref[...]
    xw = jnp.concatenate([xw1_ref[f] for f in range(CH)], axis=-1)
    y = (xw + pw) * dcol
    for f in range(CH):
        y1_ref[f] = y[:, CW * f:CW * (f + 1)]


def _tc_k1(xw1_c, degp3, poss2, w1r0):
    return pl.pallas_call(
        _tc_k1_body,
        grid=(NBLK,),
        in_specs=[
            pl.BlockSpec((CH, 512, CW), lambda i: (0, i, 0)),
            pl.BlockSpec((NC, 512, CW), lambda i: (0, i, 0)),
            pl.BlockSpec((1, 1, 512), lambda i: (i, 0, 0)),
            pl.BlockSpec((1, HID), lambda i: (0, 0)),
        ],
        out_specs=[
            pl.BlockSpec((CH, 512, CW), lambda i: (0, i, 0)),
            pl.BlockSpec((1, 1, 512), lambda i: (i, 0, 0)),
        ],
        out_shape=[
            jax.ShapeDtypeStruct((CH, N_PAD, CW), _f32),
            jax.ShapeDtypeStruct((NBLK, 1, 512), _f32),
        ],
    )(xw1_c, degp3, poss2, w1r0)


def _tc_mid_body(s_ref, dinv_ref, b_ref, g_ref, be_ref, w_ref,
                 out_ref, acc_s, acc_q, zbuf):
    p = pl.program_id(0)
    i = pl.program_id(1)

    @pl.when(jnp.logical_and(p == 0, i == 0))
    def _init():
        acc_s[...] = jnp.zeros((1, HID), _f32)
        acc_q[...] = jnp.zeros((1, HID), _f32)

    dcol = dinv_ref[...].reshape(512, 1)

    @pl.when(p == 0)
    def _stats():
        zs = jnp.concatenate([s_ref[f] for f in range(CH)], axis=-1)
        z = dcol * zs + b_ref[...]
        zbuf[i] = z
        gid = i * 512 + lax.broadcasted_iota(_i32, (512, 1), 0)
        m = gid < N
        zm = jnp.where(m, z, 0.0)
        zq = jnp.where(m, z * z, 0.0)
        acc_s[...] += jnp.sum(zm, axis=0, keepdims=True)
        acc_q[...] += jnp.sum(zq, axis=0, keepdims=True)

    @pl.when(p == 1)
    def _apply():
        z = zbuf[i]
        mean = acc_s[...] / N
        var = acc_q[...] / N - mean * mean
        a = g_ref[...] / jnp.sqrt(var + 1e-5)
        cc = be_ref[...] - mean * a
        h = jnp.maximum(z * a + cc, 0.0)
        yn = jnp.dot(h, w_ref[...], preferred_element_type=_f32) * dcol
        for f in range(CH):
            out_ref[f] = yn[:, CW * f:CW * (f + 1)]


def _tc_mid(s_c, dinv2, b, g, be, w):
    return pl.pallas_call(
        _tc_mid_body,
        grid=(2, NBLK),
        in_specs=[
            pl.BlockSpec((CH, 512, CW), lambda p, i: (0, i * (1 - p), 0)),
            pl.BlockSpec((1, 1, 512), lambda p, i: (i, 0, 0)),
            pl.BlockSpec((1, HID), lambda p, i: (0, 0)),
            pl.BlockSpec((1, HID), lambda p, i: (0, 0)),
            pl.BlockSpec((1, HID), lambda p, i: (0, 0)),
            pl.BlockSpec((HID, HID), lambda p, i: (0, 0)),
        ],
        out_specs=pl.BlockSpec((CH, 512, CW), lambda p, i: (0, i, 0)),
        out_shape=jax.ShapeDtypeStruct((CH, N_PAD, CW), _f32),
        compiler_params=pltpu.CompilerParams(
            vmem_limit_bytes=100 * 1024 * 1024),
        scratch_shapes=[pltpu.VMEM((1, HID), _f32), pltpu.VMEM((1, HID), _f32),
                        pltpu.VMEM((NBLK, 512, HID), _f32)],
    )(s_c, dinv2, b, g, be, w)


def _tc_final_body(s_ref, dinv_ref, b_ref, g_ref, be_ref,
                   batch_ref, wfc_ref, bfc_ref,
                   out_ref, acc_s, acc_q, pooled, cnt):
    p = pl.program_id(0)
    i = pl.program_id(1)

    @pl.when(jnp.logical_and(p == 0, i == 0))
    def _init():
        acc_s[...] = jnp.zeros((1, HID), _f32)
        acc_q[...] = jnp.zeros((1, HID), _f32)
        pooled[...] = jnp.zeros((G, HID), _f32)
        cnt[...] = jnp.zeros((1, G), _f32)

    dcol = dinv_ref[...].reshape(512, 1)
    zs = jnp.concatenate([s_ref[f] for f in range(CH)], axis=-1)
    z = dcol * zs + b_ref[...]

    @pl.when(p == 0)
    def _stats():
        gid = i * 512 + lax.broadcasted_iota(_i32, (512, 1), 0)
        m = gid < N
        zm = jnp.where(m, z, 0.0)
        zq = jnp.where(m, z * z, 0.0)
        acc_s[...] += jnp.sum(zm, axis=0, keepdims=True)
        acc_q[...] += jnp.sum(zq, axis=0, keepdims=True)

    @pl.when(p == 1)
    def _pool():
        mean = acc_s[...] / N
        var = acc_q[...] / N - mean * mean
        a = g_ref[...] / jnp.sqrt(var + 1e-5)
        cc = be_ref[...] - mean * a
        gid = i * 512 + lax.broadcasted_iota(_i32, (512, 1), 0)
        h = jnp.where(gid < N, jnp.maximum(z * a + cc, 0.0), 0.0)
        bcol = batch_ref[...].reshape(512, 1)
        segs = lax.broadcasted_iota(_i32, (512, G), 1)
        oh = (segs == bcol).astype(_f32)
        pooled[...] += lax.dot_general(oh, h, (((0,), (0,)), ((), ())),
                                       preferred_element_type=_f32)
        cnt[...] += jnp.sum(oh, axis=0, keepdims=True)

    @pl.when(jnp.logical_and(p == 1, i == NBLK - 1))
    def _fc():
        ccol = cnt[...].reshape(G, 1)
        meanp = pooled[...] / jnp.maximum(ccol, 1.0)
        out_ref[...] = (jnp.dot(meanp, wfc_ref[...],
                                preferred_element_type=_f32) + bfc_ref[...])


def _tc_final(s_c, dinv2, b, g, be, batch2, wfc_p, bfc_p):
    return pl.pallas_call(
        _tc_final_body,
        grid=(2, NBLK),
        in_specs=[
            pl.BlockSpec((CH, 512, CW), lambda p, i: (0, i, 0)),
            pl.BlockSpec((1, 1, 512), lambda p, i: (i, 0, 0)),
            pl.BlockSpec((1, HID), lambda p, i: (0, 0)),
            pl.BlockSpec((1, HID), lambda p, i: (0, 0)),
            pl.BlockSpec((1, HID), lambda p, i: (0, 0)),
            pl.BlockSpec((1, 1, 512), lambda p, i: (i, 0, 0)),
            pl.BlockSpec((HID, HID), lambda p, i: (0, 0)),
            pl.BlockSpec((1, HID), lambda p, i: (0, 0)),
        ],
        out_specs=pl.BlockSpec((G, HID), lambda p, i: (0, 0)),
        out_shape=jax.ShapeDtypeStruct((G, HID), _f32),
        scratch_shapes=[
            pltpu.VMEM((1, HID), _f32),
            pltpu.VMEM((1, HID), _f32),
            pltpu.VMEM((G, HID), _f32),
            pltpu.VMEM((1, G), _f32),
        ],
    )(s_c, dinv2, b, g, be, batch2, wfc_p, bfc_p)


def kernel(x, edge_index, batch, emb0, emb1, emb2, emb5,
           W1, b1, W2, b2, W3, b3, g1, be1, g2, be2, g3, be3, Wfc, bfc):
    xp = jnp.pad(x, ((0, N_PAD - N), (0, 0)))
    src = edge_index[0]
    dst = edge_index[1]
    loop = jnp.arange(N, dtype=_i32)
    src1 = jnp.concatenate([src, loop, jnp.zeros((E_PAD - E - N,), _i32)])
    dst1 = jnp.concatenate([dst, loop, jnp.full((E_PAD - E - N,), N, _i32)])
    batch2 = jnp.concatenate([batch, jnp.full((N_PAD - N,), G, _i32)]
                             ).reshape(NBLK, 1, 512)
    poss2 = xp[:, 0].reshape(NBLK, 1, 512)
    z2d = jnp.zeros((NACC_T, CW), _f32)
    ones2d = jnp.ones((EBLK, CW), _f32)

    embbd = jnp.zeros((TROWS, HID), _f32)
    embbd = embbd.at[0:96, 0:32].set(emb0)
    embbd = embbd.at[96:192, 32:64].set(emb1)
    embbd = embbd.at[192:288, 64:96].set(emb2)
    embbd = embbd.at[288:288 + 182, 96:128].set(emb5)
    w1e = W1[1:129]
    w1r0 = W1[0:1]
    wfc_p = jnp.pad(Wfc, ((0, 0), (0, HID - Wfc.shape[1])))
    bfc_p = jnp.pad(bfc, (0, HID - bfc.shape[0])).reshape(1, HID)

    tcat_c = _tc_prep(embbd, w1e)
    xw1_c, degp = _sc_emb_deg(xp, dst1, tcat_c, z2d, ones2d)
    y1_c, dinv2 = _tc_k1(xw1_c, degp, poss2, w1r0)

    s1_c = _sc_edges(y1_c, src1, dst1, z2d)
    y2_c = _tc_mid(s1_c, dinv2, b1.reshape(1, HID), g1.reshape(1, HID),
                   be1.reshape(1, HID), W2)
    s2_c = _sc_edges(y2_c, src1, dst1, z2d)
    y3_c = _tc_mid(s2_c, dinv2, b2.reshape(1, HID), g2.reshape(1, HID),
                   be2.reshape(1, HID), W3)
    s3_c = _sc_edges(y3_c, src1, dst1, z2d)
    outp = _tc_final(s3_c, dinv2, b3.reshape(1, HID), g3.reshape(1, HID),
                     be3.reshape(1, HID), batch2, wfc_p, bfc_p)
    return outp[:, :3]


# speedup vs baseline: 6.2883x; 1.4410x over previous
import jax
import jax.numpy as jnp
from jax import lax
from jax.experimental import pallas as pl
from jax.experimental.pallas import tpu as pltpu
from jax.experimental.pallas import tpu_sc as plsc

N = 50000
E = 800000
G = 512
HID = 128

NC = 2
NS = 16
LANES = 128

N_PAD = 50176
NPT = N_PAD // NS
E_PAD = 860160
EPT = E_PAD // NS
EBLK = 1120
DPT = E_PAD // NC // NS
DPAIRS = DPT // EBLK // 2
ARPT = NPT * 4 // LANES
AIB = 7
ABLK = AIB * LANES
APAIRS = ARPT // (2 * AIB)
NBLK = N_PAD // 512
CH = 16
CW = 8
CPC = CH // NC
N_ACC = 50016
CH2 = 8
CW2 = 16
CPC2 = CH2 // NC
NACC_T = N_ACC // NS
TROWS = 480

_f32 = jnp.float32
_i32 = jnp.int32


def _sc_emb_deg_body(x_hbm, dst_hbm, tcat_hbm, z2d_hbm, ones_hbm,
                     xw1_hbm, degp_hbm,
                     xbuf, gidx, nidx, dbuf0, dbuf1, obuf, gbuf0, gbuf1,
                     acc, sem_g, sem_s):
    c = lax.axis_index("c")
    s = lax.axis_index("s")
    nbase = s * NPT

    pltpu.sync_copy(ones_hbm, obuf)

    pltpu.sync_copy(x_hbm.at[pl.ds(nbase, NPT)], xbuf)
    lane = lax.iota(_i32, 16)

    def build_row(r):
        for half in range(2):
            rows16 = (2 * r + half) * 16 + lane
            nid = jnp.minimum(nbase + rows16, N)
            for t, (col, off) in enumerate(((1, 0), (2, 96), (3, 192), (4, 288))):
                vals = plsc.load_gather(xbuf, [rows16, jnp.full((16,), col, _i32)])
                iv = vals.astype(_i32) + off
                gidx[pl.ds(r * LANES + 32 * t + 16 * half, 16)] = iv
                nidx[pl.ds(r * LANES + 32 * t + 16 * half, 16)] = nid

    pl.loop(0, ARPT)(build_row)

    pltpu.sync_copy(z2d_hbm, acc.at[pl.ds(s * NACC_T, NACC_T)])
    plsc.subcore_barrier()
    dbase = (c * NS + s) * DPT

    def deg_pair(k):
        for par, db in ((0, dbuf0), (1, dbuf1)):
            b = 2 * k + par

            @pl.when(k >= 1)
            def _drain():
                pltpu.make_async_copy(ones_hbm, obuf, sem_s).wait()

            pltpu.sync_copy(dst_hbm.at[pl.ds(dbase + b * EBLK, EBLK)], db)
            pltpu.async_copy(obuf, acc.at[db], sem_s, add=True)

    pl.loop(0, DPAIRS)(deg_pair)
    for _ in range(2):
        pltpu.make_async_copy(ones_hbm, obuf, sem_s).wait()
    plsc.subcore_barrier()
    pltpu.sync_copy(acc.at[pl.ds(s * NACC_T, NACC_T)],
                    degp_hbm.at[c, pl.ds(s * NACC_T, NACC_T)])

    for fc in range(CPC):
        f = CPC * c + fc
        pltpu.sync_copy(z2d_hbm, acc.at[pl.ds(s * NACC_T, NACC_T)])
        plsc.subcore_barrier()

        def chunk_pair(k, f=f):
            for par, gb in ((0, gbuf0), (1, gbuf1)):
                b = 2 * k + par

                @pl.when(k >= 1)
                def _drain():
                    pltpu.make_async_copy(
                        xw1_hbm.at[0, pl.ds(0, ABLK)], gb, sem_s).wait()

                pltpu.async_copy(
                    tcat_hbm.at[f].at[gidx.at[pl.ds(b * ABLK, ABLK)]],
                    gb, sem_g).wait()
                pltpu.async_copy(gb, acc.at[nidx.at[pl.ds(b * ABLK, ABLK)]],
                                 sem_s, add=True)

        pl.loop(0, APAIRS)(chunk_pair)
        for gb in (gbuf0, gbuf1):
            pltpu.make_async_copy(
                xw1_hbm.at[0, pl.ds(0, ABLK)], gb, sem_s).wait()
        plsc.subcore_barrier()
        pltpu.sync_copy(acc.at[pl.ds(s * NACC_T, NACC_T)],
                        xw1_hbm.at[f, pl.ds(s * NACC_T, NACC_T)])


def _sc_emb_deg(xp, dst1, tcat_c, z2d, ones2d):
    mesh = plsc.VectorSubcoreMesh(core_axis_name="c", subcore_axis_name="s")
    return pl.kernel(
        _sc_emb_deg_body,
        out_type=(jax.ShapeDtypeStruct((CH, N_PAD, CW), _f32),
                  jax.ShapeDtypeStruct((NC, N_PAD, CW), _f32)),
        mesh=mesh,
        compiler_params=pltpu.CompilerParams(needs_layout_passes=False,
                                             use_tc_tiling_on_sc=False),
        scratch_types=[
            pltpu.VMEM((NPT, 5), _f32),
            pltpu.VMEM((ARPT * LANES,), _i32),
            pltpu.VMEM((ARPT * LANES,), _i32),
            pltpu.VMEM((EBLK,), _i32),
            pltpu.VMEM((EBLK,), _i32),
            pltpu.VMEM((EBLK, CW), _f32),
            pltpu.VMEM((ABLK, CW), _f32),
            pltpu.VMEM((ABLK, CW), _f32),
            pltpu.VMEM_SHARED((N_ACC, CW), _f32),
            pltpu.SemaphoreType.DMA,
            pltpu.SemaphoreType.DMA,
        ],
    )(xp, dst1, tcat_c, z2d, ones2d)


E2BLK = 640
E2NB = EPT // E2BLK
E2PAIRS = E2NB // 2


def _sc_edges_body(y_hbm, src_hbm, dst_hbm, z16_hbm, s_hbm,
                   sbuf0, sbuf1, dbuf0, dbuf1, gbuf0, gbuf1,
                   acc, sem_g, sem_s, sem_i):
    c = lax.axis_index("c")
    s = lax.axis_index("s")
    ebase = s * EPT

    for fc in range(CPC2):
        f = CPC2 * c + fc
        pltpu.sync_copy(z16_hbm, acc.at[pl.ds(s * NACC_T, NACC_T)])
        plsc.subcore_barrier()

        pltpu.async_copy(src_hbm.at[pl.ds(ebase, E2BLK)], sbuf0, sem_i)
        pltpu.async_copy(dst_hbm.at[pl.ds(ebase, E2BLK)], dbuf0, sem_i)

        def edge_pair(k, f=f):
            for par, (sb, db, gb, sbn, dbn) in (
                    (0, (sbuf0, dbuf0, gbuf0, sbuf1, dbuf1)),
                    (1, (sbuf1, dbuf1, gbuf1, sbuf0, dbuf0))):
                b = 2 * k + par
                pltpu.make_async_copy(src_hbm.at[pl.ds(0, E2BLK)], sb,
                                      sem_i).wait()
                pltpu.make_async_copy(dst_hbm.at[pl.ds(0, E2BLK)], db,
                                      sem_i).wait()

                @pl.when(k >= 1)
                def _drain():
                    pltpu.make_async_copy(
                        y_hbm.at[0, pl.ds(0, E2BLK)], gb, sem_s).wait()

                if par == 0:
                    nxt = ebase + (b + 1) * E2BLK
                    pltpu.async_copy(src_hbm.at[pl.ds(nxt, E2BLK)], sbn, sem_i)
                    pltpu.async_copy(dst_hbm.at[pl.ds(nxt, E2BLK)], dbn, sem_i)
                else:
                    @pl.when(k + 1 < E2PAIRS)
                    def _pref():
                        nxt = ebase + (b + 1) * E2BLK
                        pltpu.async_copy(src_hbm.at[pl.ds(nxt, E2BLK)], sbn,
                                         sem_i)
                        pltpu.async_copy(dst_hbm.at[pl.ds(nxt, E2BLK)], dbn,
                                         sem_i)

                pltpu.async_copy(y_hbm.at[f].at[sb], gb, sem_g).wait()
                pltpu.async_copy(gb, acc.at[db], sem_s, add=True)

        pl.loop(0, E2PAIRS)(edge_pair)
        for gb in (gbuf0, gbuf1):
            pltpu.make_async_copy(
                y_hbm.at[0, pl.ds(0, E2BLK)], gb, sem_s).wait()
        plsc.subcore_barrier()
        pltpu.sync_copy(acc.at[pl.ds(s * NACC_T, NACC_T)],
                        s_hbm.at[f, pl.ds(s * NACC_T, NACC_T)])


def _sc_edges(y_c, src1, dst1, z16):
    mesh = plsc.VectorSubcoreMesh(core_axis_name="c", subcore_axis_name="s")
    return pl.kernel(
        _sc_edges_body,
        out_type=jax.ShapeDtypeStruct((CH2, N_PAD, CW2), _f32),
        mesh=mesh,
        compiler_params=pltpu.CompilerParams(use_tc_tiling_on_sc=False),
        scratch_types=[
            pltpu.VMEM((E2BLK,), _i32),
            pltpu.VMEM((E2BLK,), _i32),
            pltpu.VMEM((E2BLK,), _i32),
            pltpu.VMEM((E2BLK,), _i32),
            pltpu.VMEM((E2BLK, CW2), _f32),
            pltpu.VMEM((E2BLK, CW2), _f32),
            pltpu.VMEM_SHARED((N_ACC, CW2), _f32),
            pltpu.SemaphoreType.DMA,
            pltpu.SemaphoreType.DMA,
            pltpu.SemaphoreType.DMA,
        ],
    )(y_c, src1, dst1, z16)


def _tc_prep_body(embbd_ref, w1e_ref, out_ref):
    t = jnp.dot(embbd_ref[...], w1e_ref[...], preferred_element_type=_f32)
    for f in range(CH):
        out_ref[f] = t[:, CW * f:CW * (f + 1)]


def _tc_prep(embbd, w1e):
    return pl.pallas_call(
        _tc_prep_body,
        out_shape=jax.ShapeDtypeStruct((CH, TROWS, CW), _f32),
    )(embbd, w1e)


def _tc_k1_body(xw1_ref, degp_ref, poss_ref, w1r0_ref, y1_ref, dinv_ref):
    deg = degp_ref[0][:, 0:1] + degp_ref[1][:, 0:1]
    dcol = 1.0 / jnp.sqrt(deg)
    dinv_ref[0] = dcol.reshape(1, 512)
    pcol = poss_ref[...].reshape(512, 1)
    pw = pcol * w1r0_ref[...]
    xw = jnp.concatenate([xw1_ref[f] for f in range(CH)], axis=-1)
    y = (xw + pw) * dcol
    for f in range(CH2):
        y1_ref[f] = y[:, CW2 * f:CW2 * (f + 1)]


def _tc_k1(xw1_c, degp3, poss2, w1r0):
    return pl.pallas_call(
        _tc_k1_body,
        grid=(NBLK,),
        in_specs=[
            pl.BlockSpec((CH, 512, CW), lambda i: (0, i, 0)),
            pl.BlockSpec((NC, 512, CW), lambda i: (0, i, 0)),
            pl.BlockSpec((1, 1, 512), lambda i: (i, 0, 0)),
            pl.BlockSpec((1, HID), lambda i: (0, 0)),
        ],
        out_specs=[
            pl.BlockSpec((CH2, 512, CW2), lambda i: (0, i, 0)),
            pl.BlockSpec((1, 1, 512), lambda i: (i, 0, 0)),
        ],
        out_shape=[
            jax.ShapeDtypeStruct((CH2, N_PAD, CW2), _f32),
            jax.ShapeDtypeStruct((NBLK, 1, 512), _f32),
        ],
    )(xw1_c, degp3, poss2, w1r0)


def _tc_mid_body(s_ref, dinv_ref, b_ref, g_ref, be_ref, w_ref,
                 out_ref, acc_s, acc_q, zbuf):
    p = pl.program_id(0)
    i = pl.program_id(1)

    @pl.when(jnp.logical_and(p == 0, i == 0))
    def _init():
        acc_s[...] = jnp.zeros((1, HID), _f32)
        acc_q[...] = jnp.zeros((1, HID), _f32)

    dcol = dinv_ref[...].reshape(512, 1)

    @pl.when(p == 0)
    def _stats():
        zs = jnp.concatenate([s_ref[f] for f in range(CH2)], axis=-1)
        z = dcol * zs + b_ref[...]
        zbuf[i] = z
        gid = i * 512 + lax.broadcasted_iota(_i32, (512, 1), 0)
        m = gid < N
        zm = jnp.where(m, z, 0.0)
        zq = jnp.where(m, z * z, 0.0)
        acc_s[...] += jnp.sum(zm, axis=0, keepdims=True)
        acc_q[...] += jnp.sum(zq, axis=0, keepdims=True)

    @pl.when(p == 1)
    def _apply():
        z = zbuf[i]
        mean = acc_s[...] / N
        var = acc_q[...] / N - mean * mean
        a = g_ref[...] / jnp.sqrt(var + 1e-5)
        cc = be_ref[...] - mean * a
        h = jnp.maximum(z * a + cc, 0.0)
        yn = jnp.dot(h, w_ref[...], preferred_element_type=_f32) * dcol
        for f in range(CH2):
            out_ref[f] = yn[:, CW2 * f:CW2 * (f + 1)]


def _tc_mid(s_c, dinv2, b, g, be, w):
    return pl.pallas_call(
        _tc_mid_body,
        grid=(2, NBLK),
        in_specs=[
            pl.BlockSpec((CH2, 512, CW2), lambda p, i: (0, i * (1 - p), 0)),
            pl.BlockSpec((1, 1, 512), lambda p, i: (i, 0, 0)),
            pl.BlockSpec((1, HID), lambda p, i: (0, 0)),
            pl.BlockSpec((1, HID), lambda p, i: (0, 0)),
            pl.BlockSpec((1, HID), lambda p, i: (0, 0)),
            pl.BlockSpec((HID, HID), lambda p, i: (0, 0)),
        ],
        out_specs=pl.BlockSpec((CH2, 512, CW2), lambda p, i: (0, i, 0)),
        out_shape=jax.ShapeDtypeStruct((CH2, N_PAD, CW2), _f32),
        compiler_params=pltpu.CompilerParams(
            vmem_limit_bytes=100 * 1024 * 1024),
        scratch_shapes=[pltpu.VMEM((1, HID), _f32), pltpu.VMEM((1, HID), _f32),
                        pltpu.VMEM((NBLK, 512, HID), _f32)],
    )(s_c, dinv2, b, g, be, w)


def _tc_final_body(s_ref, dinv_ref, b_ref, g_ref, be_ref,
                   batch_ref, wfc_ref, bfc_ref,
                   out_ref, acc_s, acc_q, pooled, cnt):
    p = pl.program_id(0)
    i = pl.program_id(1)

    @pl.when(jnp.logical_and(p == 0, i == 0))
    def _init():
        acc_s[...] = jnp.zeros((1, HID), _f32)
        acc_q[...] = jnp.zeros((1, HID), _f32)
        pooled[...] = jnp.zeros((G, HID), _f32)
        cnt[...] = jnp.zeros((1, G), _f32)

    dcol = dinv_ref[...].reshape(512, 1)
    zs = jnp.concatenate([s_ref[f] for f in range(CH2)], axis=-1)
    z = dcol * zs + b_ref[...]

    @pl.when(p == 0)
    def _stats():
        gid = i * 512 + lax.broadcasted_iota(_i32, (512, 1), 0)
        m = gid < N
        zm = jnp.where(m, z, 0.0)
        zq = jnp.where(m, z * z, 0.0)
        acc_s[...] += jnp.sum(zm, axis=0, keepdims=True)
        acc_q[...] += jnp.sum(zq, axis=0, keepdims=True)

    @pl.when(p == 1)
    def _pool():
        mean = acc_s[...] / N
        var = acc_q[...] / N - mean * mean
        a = g_ref[...] / jnp.sqrt(var + 1e-5)
        cc = be_ref[...] - mean * a
        gid = i * 512 + lax.broadcasted_iota(_i32, (512, 1), 0)
        h = jnp.where(gid < N, jnp.maximum(z * a + cc, 0.0), 0.0)
        bcol = batch_ref[...].reshape(512, 1)
        segs = lax.broadcasted_iota(_i32, (512, G), 1)
        oh = (segs == bcol).astype(_f32)
        pooled[...] += lax.dot_general(oh, h, (((0,), (0,)), ((), ())),
                                       preferred_element_type=_f32)
        cnt[...] += jnp.sum(oh, axis=0, keepdims=True)

    @pl.when(jnp.logical_and(p == 1, i == NBLK - 1))
    def _fc():
        ccol = cnt[...].reshape(G, 1)
        meanp = pooled[...] / jnp.maximum(ccol, 1.0)
        out_ref[...] = (jnp.dot(meanp, wfc_ref[...],
                                preferred_element_type=_f32) + bfc_ref[...])


def _tc_final(s_c, dinv2, b, g, be, batch2, wfc_p, bfc_p):
    return pl.pallas_call(
        _tc_final_body,
        grid=(2, NBLK),
        in_specs=[
            pl.BlockSpec((CH2, 512, CW2), lambda p, i: (0, i, 0)),
            pl.BlockSpec((1, 1, 512), lambda p, i: (i, 0, 0)),
            pl.BlockSpec((1, HID), lambda p, i: (0, 0)),
            pl.BlockSpec((1, HID), lambda p, i: (0, 0)),
            pl.BlockSpec((1, HID), lambda p, i: (0, 0)),
            pl.BlockSpec((1, 1, 512), lambda p, i: (i, 0, 0)),
            pl.BlockSpec((HID, HID), lambda p, i: (0, 0)),
            pl.BlockSpec((1, HID), lambda p, i: (0, 0)),
        ],
        out_specs=pl.BlockSpec((G, HID), lambda p, i: (0, 0)),
        out_shape=jax.ShapeDtypeStruct((G, HID), _f32),
        scratch_shapes=[
            pltpu.VMEM((1, HID), _f32),
            pltpu.VMEM((1, HID), _f32),
            pltpu.VMEM((G, HID), _f32),
            pltpu.VMEM((1, G), _f32),
        ],
    )(s_c, dinv2, b, g, be, batch2, wfc_p, bfc_p)


def kernel(x, edge_index, batch, emb0, emb1, emb2, emb5,
           W1, b1, W2, b2, W3, b3, g1, be1, g2, be2, g3, be3, Wfc, bfc):
    xp = jnp.pad(x, ((0, N_PAD - N), (0, 0)))
    src = edge_index[0]
    dst = edge_index[1]
    loop = jnp.arange(N, dtype=_i32)
    src1 = jnp.concatenate([src, loop, jnp.zeros((E_PAD - E - N,), _i32)])
    dst1 = jnp.concatenate([dst, loop, jnp.full((E_PAD - E - N,), N, _i32)])
    batch2 = jnp.concatenate([batch, jnp.full((N_PAD - N,), G, _i32)]
                             ).reshape(NBLK, 1, 512)
    poss2 = xp[:, 0].reshape(NBLK, 1, 512)
    z2d = jnp.zeros((NACC_T, CW), _f32)
    z16 = jnp.zeros((NACC_T, CW2), _f32)
    ones2d = jnp.ones((EBLK, CW), _f32)

    embbd = jnp.zeros((TROWS, HID), _f32)
    embbd = embbd.at[0:96, 0:32].set(emb0)
    embbd = embbd.at[96:192, 32:64].set(emb1)
    embbd = embbd.at[192:288, 64:96].set(emb2)
    embbd = embbd.at[288:288 + 182, 96:128].set(emb5)
    w1e = W1[1:129]
    w1r0 = W1[0:1]
    wfc_p = jnp.pad(Wfc, ((0, 0), (0, HID - Wfc.shape[1])))
    bfc_p = jnp.pad(bfc, (0, HID - bfc.shape[0])).reshape(1, HID)

    tcat_c = _tc_prep(embbd, w1e)
    xw1_c, degp = _sc_emb_deg(xp, dst1, tcat_c, z2d, ones2d)
    y1_c, dinv2 = _tc_k1(xw1_c, degp, poss2, w1r0)

    s1_c = _sc_edges(y1_c, src1, dst1, z16)
    y2_c = _tc_mid(s1_c, dinv2, b1.reshape(1, HID), g1.reshape(1, HID),
                   be1.reshape(1, HID), W2)
    s2_c = _sc_edges(y2_c, src1, dst1, z16)
    y3_c = _tc_mid(s2_c, dinv2, b2.reshape(1, HID), g2.reshape(1, HID),
                   be2.reshape(1, HID), W3)
    s3_c = _sc_edges(y3_c, src1, dst1, z16)
    outp = _tc_final(s3_c, dinv2, b3.reshape(1, HID), g3.reshape(1, HID),
                     be3.reshape(1, HID), batch2, wfc_p, bfc_p)
    return outp[:, :3]
